# baseline (speedup 1.0000x reference)
import functools
import math

import jax
import jax.numpy as jnp
from jax import lax
from jax.experimental import pallas as pl
from jax.experimental.pallas import tpu as pltpu
from jax.experimental.pallas import tpu_sc as plsc

_NC = 2
_NS = 16
_LANES = 16
_NW = _NC * _NS

_NBUF = 2


@functools.lru_cache(maxsize=None)
def _build(b_sz, l_sz, v, d, scale):
    assert b_sz % (_NW * 128) == 0 and d % _LANES == 0
    n = b_sz * l_sz
    per_w = n // _NW
    assert per_w == 128 * l_sz
    cb_n = d // 8
    mesh = plsc.VectorSubcoreMesh(core_axis_name="c", subcore_axis_name="s")

    @functools.partial(
        pl.kernel,
        out_type=jax.ShapeDtypeStruct((l_sz, cb_n, _NW, 8, 128), jnp.float32),
        mesh=mesh,
        scratch_types=[
            pltpu.VMEM((per_w,), jnp.int32),
            [pltpu.VMEM((128,), jnp.int32) for _ in range(_NBUF)],
            [pltpu.VMEM((128, d), jnp.float32) for _ in range(_NBUF)],
            [pltpu.VMEM((d, 128), jnp.float32) for _ in range(_NBUF)],
            [pltpu.SemaphoreType.DMA for _ in range(_NBUF)],
            [pltpu.SemaphoreType.DMA for _ in range(_NBUF)],
        ],
        compiler_params=pltpu.CompilerParams(
            use_tc_tiling_on_sc=False, needs_layout_passes=False
        ),
    )
    def emb(table_hbm, idx_hbm, out_hbm, idx_v, cidx, rows, tbuf, gsem, osem):
        wid = lax.axis_index("s") * _NC + lax.axis_index("c")
        base = wid * per_w

        pltpu.sync_copy(idx_hbm.at[pl.ds(base, per_w)], idx_v)

        iota = lax.iota(jnp.int32, _LANES)
        stride = iota * l_sz

        def build_and_gather(step, b):
            for g in range(8):
                pos = stride + (g * _LANES * l_sz + step)
                vals = plsc.load_gather(idx_v, [pos])
                cidx[b][pl.ds(g * _LANES, _LANES)] = vals
            pltpu.async_copy(table_hbm.at[cidx[b]], rows[b], gsem[b])

        for b in range(_NBUF):
            build_and_gather(b, b)

        @pl.loop(0, l_sz, step=_NBUF)
        def _(l0):
            for b in range(_NBUF):
                step = l0 + b
                pltpu.make_async_copy(
                    table_hbm.at[pl.ds(0, 128)], rows[b], gsem[b]
                ).wait()

                @pl.when(step >= _NBUF)
                def _():
                    for cb in range(cb_n):
                        pltpu.make_async_copy(
                            tbuf[b].at[pl.ds(cb * 8, 8)],
                            out_hbm.at[0, cb, wid],
                            osem[b],
                        ).wait()

                @pl.loop(0, d)
                def _(c):
                    cvec = jnp.full((_LANES,), c, jnp.int32)
                    for g in range(8):
                        tok = iota + g * _LANES
                        vals = plsc.load_gather(rows[b], [tok, cvec])
                        tbuf[b][c, pl.ds(g * _LANES, _LANES)] = vals * scale

                for cb in range(cb_n):
                    pltpu.async_copy(
                        tbuf[b].at[pl.ds(cb * 8, 8)],
                        out_hbm.at[step, cb, wid],
                        osem[b],
                    )

                @pl.when(step + _NBUF < l_sz)
                def _():
                    build_and_gather(step + _NBUF, b)

        for b in range(_NBUF):
            for cb in range(cb_n):
                pltpu.make_async_copy(
                    tbuf[b].at[pl.ds(cb * 8, 8)],
                    out_hbm.at[0, cb, wid],
                    osem[b],
                ).wait()

    return emb


def kernel(x, table):
    b_sz, l_sz = x.shape
    v, d = table.shape
    idx = x.reshape(-1).astype(jnp.int32)
    out5 = _build(b_sz, l_sz, v, d, math.sqrt(d))(table, idx)
    return out5.transpose(2, 4, 0, 1, 3).reshape(b_sz, l_sz, d)
